# Initial kernel scaffold; baseline (speedup 1.0000x reference)
import jax
import jax.numpy as jnp
from jax.experimental import pallas as pl


def kernel(inputs, W, b):
    raise NotImplementedError("write your pallas kernel here")


# speedup vs baseline: 1.1039x; 1.1039x over previous
import jax
import jax.numpy as jnp
from jax.experimental import pallas as pl

_D = 4096
_E = 64
_K = 8
_S = 2
_BLOCK = 512


def _gate_kernel(x_ref, w_ref, b_ref, probs_ref, idx_ref, logits_ref):
    x = x_ref[...]
    w = w_ref[...]
    b = b_ref[...]
    logits = jnp.dot(x, w, preferred_element_type=jnp.float32) + b
    logits_ref[...] = logits

    bsz = logits.shape[0]
    col = jax.lax.broadcasted_iota(jnp.int32, (bsz, _E), 1)
    work = jnp.where(col >= _S, logits, -jnp.inf)
    vals, idxs = [], []
    for _ in range(_K):
        m = jnp.max(work, axis=1, keepdims=True)
        im = jnp.min(jnp.where(work == m, col, _E), axis=1, keepdims=True)
        vals.append(m)
        idxs.append(im)
        work = jnp.where(col == im, -jnp.inf, work)

    topk_vals = jnp.concatenate([logits[:, :_S]] + vals, axis=1)
    shared_idx = jax.lax.broadcasted_iota(jnp.int32, (bsz, _S), 1)
    topk_idx = jnp.concatenate([shared_idx] + idxs, axis=1)

    mx = jnp.max(topk_vals, axis=1, keepdims=True)
    e = jnp.exp(topk_vals - mx)
    probs_ref[...] = e / jnp.sum(e, axis=1, keepdims=True)
    idx_ref[...] = topk_idx


def kernel(inputs, W, b):
    n = inputs.shape[0]
    grid = (n // _BLOCK,)
    probs, idx, logits = pl.pallas_call(
        _gate_kernel,
        grid=grid,
        in_specs=[
            pl.BlockSpec((_BLOCK, _D), lambda i: (i, 0)),
            pl.BlockSpec((_D, _E), lambda i: (0, 0)),
            pl.BlockSpec((1, _E), lambda i: (0, 0)),
        ],
        out_specs=[
            pl.BlockSpec((_BLOCK, _S + _K), lambda i: (i, 0)),
            pl.BlockSpec((_BLOCK, _S + _K), lambda i: (i, 0)),
            pl.BlockSpec((_BLOCK, _E), lambda i: (i, 0)),
        ],
        out_shape=[
            jax.ShapeDtypeStruct((n, _S + _K), jnp.float32),
            jax.ShapeDtypeStruct((n, _S + _K), jnp.int32),
            jax.ShapeDtypeStruct((n, _E), jnp.float32),
        ],
    )(inputs, W.astype(jnp.float32), b.reshape(1, _E))
    return probs, idx, logits
